# baseline (speedup 1.0000x reference)
import functools

import jax
import jax.numpy as jnp
from jax import lax
from jax.experimental import pallas as pl
from jax.experimental.pallas import tpu as pltpu
from jax.experimental.pallas import tpu_sc as plsc

N_NODES = 10000
N_EDGES = 320000
NC = 2
NS = 16
NW = NC * NS
CH = 128
GCH = 4
GROUP = GCH * CH
GPR0 = 16
REPS0 = 2
GPR1 = 8
REPS1 = 1
G0 = GPR0 * REPS0
G1 = GPR1 * REPS1
NBUF = 2
GPRMAX = max(GPR0, GPR1)
TOT = NS * (G0 + G1) * GROUP
NCHD = TOT // CH
DEG_G = NCHD // NW
NROWS = 10240
RPT = NROWS // NS
DEG_W = 16
H = 64


def _deg_body(dstc_hbm, ones_hbm, zeros_hbm, out_hbm, didx, onev, accum):
    c = lax.axis_index("c")
    s = lax.axis_index("s")
    wid = c * NS + s
    r0 = s * RPT
    pltpu.sync_copy(zeros_hbm.at[pl.ds(r0, RPT)], accum.at[pl.ds(r0, RPT)])
    pltpu.sync_copy(ones_hbm, onev)
    pltpu.sync_copy(dstc_hbm.at[pl.ds(wid * DEG_G, DEG_G)], didx)
    plsc.subcore_barrier()

    def body(j, carry):
        pltpu.sync_copy(onev, accum.at[didx.at[j]], add=True)
        return carry

    lax.fori_loop(0, DEG_G, body, 0)
    plsc.subcore_barrier()
    pltpu.sync_copy(accum.at[pl.ds(r0, RPT)], out_hbm.at[c, pl.ds(r0, RPT)])


def _scatter_body(src_hbm, dstc_hbm, vals_hbm, zeros_hbm, out_hbm,
                  sidx, didx, *rest):
    rows = rest[:NBUF]
    accum = rest[NBUF]
    sems = rest[NBUF + 1:]
    c = lax.axis_index("c")
    s = lax.axis_index("s")
    r0 = s * RPT
    pltpu.sync_copy(zeros_hbm.at[pl.ds(r0, RPT)], accum.at[pl.ds(r0, RPT)])

    def gather(j, b):
        off = pl.multiple_of(j * GROUP, GROUP)
        return pltpu.async_copy(vals_hbm.at[sidx.at[pl.ds(off, GROUP)]],
                                rows[b], sems[b])

    def gwait(b):
        pltpu.make_async_copy(zeros_hbm.at[pl.ds(0, GROUP)], rows[b],
                              sems[b]).wait()

    def scat(j, b):
        for k in range(GCH):
            pltpu.sync_copy(rows[b].at[pl.ds(k * CH, CH)],
                            accum.at[didx.at[j * GCH + k]], add=True)

    def run(gpr, reps, ebase, cbase):
        assert gpr % NBUF == 0
        for rep in range(reps):
            eb = ebase + rep * (gpr * GROUP)
            cb = cbase + rep * (gpr * GCH)
            n_ed = gpr * GROUP
            pltpu.sync_copy(
                src_hbm.at[pl.ds(pl.multiple_of(eb, GROUP), n_ed)],
                sidx.at[pl.ds(0, n_ed)])
            pltpu.sync_copy(dstc_hbm.at[pl.ds(cb, gpr * GCH)],
                            didx.at[pl.ds(0, gpr * GCH)])
            if rep == 0:
                plsc.subcore_barrier()
            for b in range(NBUF):
                gather(b, b)

            def body(i, carry):
                for b in range(NBUF):
                    j = i * NBUF + b
                    gwait(b)
                    scat(j, b)

                    @pl.when(j + NBUF < gpr)
                    def _():
                        gather(j + NBUF, b)
                return carry

            lax.fori_loop(0, gpr // NBUF, body, 0)
        plsc.subcore_barrier()
        pltpu.sync_copy(accum.at[pl.ds(r0, RPT)], out_hbm.at[c, pl.ds(r0, RPT)])

    @pl.when(c == 0)
    def _():
        run(GPR0, REPS0, s * (G0 * GROUP), s * (G0 * GCH))

    @pl.when(c == 1)
    def _():
        run(GPR1, REPS1, NS * (G0 * GROUP) + s * (G1 * GROUP),
            NS * (G0 * GCH) + s * (G1 * GCH))


@functools.lru_cache(maxsize=None)
def _build_sc_kernels():
    mesh = plsc.VectorSubcoreMesh(core_axis_name="c", subcore_axis_name="s",
                                  num_cores=NC, num_subcores=NS)
    cparams = pltpu.CompilerParams(use_tc_tiling_on_sc=False)
    deg_k = pl.kernel(
        _deg_body,
        out_type=jax.ShapeDtypeStruct((NC, NROWS, DEG_W), jnp.float32),
        mesh=mesh,
        scratch_types=[
            pltpu.VMEM((DEG_G, CH), jnp.int32),
            pltpu.VMEM((CH, DEG_W), jnp.float32),
            pltpu.VMEM_SHARED((NROWS, DEG_W), jnp.float32),
        ],
        compiler_params=cparams,
    )
    scat_k = pl.kernel(
        _scatter_body,
        out_type=jax.ShapeDtypeStruct((NC, NROWS, H), jnp.float32),
        mesh=mesh,
        scratch_types=[
            pltpu.VMEM((GPRMAX * GROUP,), jnp.int32),
            pltpu.VMEM((GPRMAX * GCH, CH), jnp.int32),
            *[pltpu.VMEM((GROUP, H), jnp.float32) for _ in range(NBUF)],
            pltpu.VMEM_SHARED((NROWS, H), jnp.float32),
            *[pltpu.SemaphoreType.DMA for _ in range(NBUF)],
        ],
        compiler_params=cparams,
    )
    return deg_k, scat_k


def _seg_scatter(src_flat, dstc, vals, zeros_h):
    return _build_sc_kernels()[1](src_flat, dstc, vals, zeros_h)


def _deg_count(dstc, ones_deg, zeros_deg):
    return _build_sc_kernels()[0](dstc, ones_deg, zeros_deg)


def _mm1_body(x_ref, w_ref, da_ref, db_ref, o_ref):
    deg = da_ref[...] + db_ref[...] + 1.0
    dinv = lax.rsqrt(deg)
    o_ref[...] = jnp.dot(x_ref[...], w_ref[...],
                         preferred_element_type=jnp.float32) * dinv


def _mid_body(sa_ref, sb_ref, hs_ref, da_ref, db_ref, b1_ref, mt_ref,
              sqt_ref, tst_ref, al_ref, gh_ref, gy_ref, bh_ref, by_ref,
              w2h_ref, w2y_ref, o_ref):
    deg2 = da_ref[...] + db_ref[...]
    dinv = lax.rsqrt(deg2 + 1.0)
    h = jnp.maximum((sa_ref[...] + sb_ref[...] + hs_ref[...]) * dinv
                    + b1_ref[...], 0.0)
    hm = jnp.dot(h, mt_ref[...], preferred_element_type=jnp.float32)
    d_feat = (jnp.sum(h * h, axis=1, keepdims=True) - 2.0 * hm + sqt_ref[...])
    deg_n = deg2 / jnp.maximum(jnp.max(deg2), 1.0)
    d_struct = (deg_n - tst_ref[...]) ** 2
    alpha = al_ref[0, 0]
    y = (1.0 - alpha) * d_feat + alpha * d_struct
    eps = 1e-5
    muh = jnp.mean(h, axis=0, keepdims=True)
    varh = jnp.mean((h - muh) ** 2, axis=0, keepdims=True)
    zh = gh_ref[...] * (h - muh) / jnp.sqrt(varh + eps) + bh_ref[...]
    muy = jnp.mean(y, axis=0, keepdims=True)
    vary = jnp.mean((y - muy) ** 2, axis=0, keepdims=True)
    zy = gy_ref[...] * (y - muy) / jnp.sqrt(vary + eps) + by_ref[...]
    ts = jnp.dot(zh, w2h_ref[...], preferred_element_type=jnp.float32)
    ts = ts + jnp.dot(zy, w2y_ref[...], preferred_element_type=jnp.float32)
    o_ref[...] = ts * dinv


def _fin_body(sa_ref, sb_ref, ts_ref, da_ref, db_ref, b2_ref, wl_ref,
              bl_ref, out_ref, z_ref):
    deg = da_ref[...] + db_ref[...] + 1.0
    dinv = lax.rsqrt(deg)
    z = jnp.maximum((sa_ref[...] + sb_ref[...] + ts_ref[...]) * dinv
                    + b2_ref[...], 0.0)
    z_ref[...] = z
    out_ref[...] = jnp.dot(z, wl_ref[...],
                           preferred_element_type=jnp.float32) + bl_ref[...]


def _mm1_call(x, W1, da, db):
    return pl.pallas_call(
        _mm1_body,
        out_shape=jax.ShapeDtypeStruct((N_NODES, H), jnp.float32),
    )(x, W1, da, db)


def _mid_call(sa, sb, hs, da, db, b1, mt, sqt, tst, al, gh, gy, bh, by,
              w2h, w2y):
    return pl.pallas_call(
        _mid_body,
        out_shape=jax.ShapeDtypeStruct((N_NODES, H), jnp.float32),
    )(sa, sb, hs, da, db, b1, mt, sqt, tst, al, gh, gy, bh, by, w2h, w2y)


def _fin_call(sa, sb, ts, da, db, b2, Wl, bl):
    return pl.pallas_call(
        _fin_body,
        out_shape=(jax.ShapeDtypeStruct((N_NODES, 16), jnp.float32),
                   jax.ShapeDtypeStruct((N_NODES, H), jnp.float32)),
    )(sa, sb, ts, da, db, b2, Wl, bl)


def kernel(x, edge_index, W1, b1, Tf, TA, qw, alpha0, gamma, beta, W2, b2,
           Wl, bl):
    pad = TOT - N_EDGES
    src_flat = jnp.concatenate([edge_index[0],
                                jnp.zeros((pad,), edge_index.dtype)])
    dst = jnp.concatenate([edge_index[1],
                           jnp.full((pad,), N_NODES, edge_index.dtype)])
    dstc = dst.reshape(NCHD, CH)
    zeros_h = jnp.zeros((NROWS, H), jnp.float32)
    zeros_deg = jnp.zeros((NROWS, DEG_W), jnp.float32)
    ones_deg = jnp.ones((CH, DEG_W), jnp.float32)

    q = jax.nn.softmax(qw, axis=1)
    alpha = jax.nn.sigmoid(alpha0)
    mean_t = jnp.einsum('tk,tkd->td', q, Tf)
    sq_t = jnp.einsum('tk,tk->t', q, jnp.sum(Tf * Tf, axis=-1))
    t_struct = jnp.einsum('tk,tkl,tl->t', q, TA, q)

    degp = _deg_count(dstc, ones_deg, zeros_deg)
    da = degp[0, :N_NODES, 0:1]
    db = degp[1, :N_NODES, 0:1]

    hs0 = _mm1_call(x, W1, da, db)
    p1 = _seg_scatter(src_flat, dstc, hs0, zeros_h)
    ts = _mid_call(p1[0, :N_NODES], p1[1, :N_NODES], hs0, da, db,
                   b1.reshape(1, H), mean_t.T, sq_t.reshape(1, -1),
                   t_struct.reshape(1, -1), alpha.reshape(1, 1),
                   gamma[:H].reshape(1, H), gamma[H:].reshape(1, -1),
                   beta[:H].reshape(1, H), beta[H:].reshape(1, -1),
                   W2[:H], W2[H:])

    p2 = _seg_scatter(src_flat, dstc, ts, zeros_h)
    out, z2 = _fin_call(p2[0, :N_NODES], p2[1, :N_NODES], ts, da, db,
                        b2.reshape(1, H), Wl, bl.reshape(1, -1))
    return (out, z2)


# speedup vs baseline: 1.8348x; 1.8348x over previous
import functools

import jax
import jax.numpy as jnp
from jax import lax
from jax.experimental import pallas as pl
from jax.experimental.pallas import tpu as pltpu
from jax.experimental.pallas import tpu_sc as plsc

N_NODES = 10000
N_EDGES = 320000
NC = 2
NS = 16
NW = NC * NS
CH = 128
GCH = 2
GROUP = GCH * CH
GPR0 = 20
REPS0 = 2
GPR1 = 20
REPS1 = 2
G0 = GPR0 * REPS0
G1 = GPR1 * REPS1
NBUF = 2
GPRMAX = max(GPR0, GPR1)
TOT = NS * (G0 + G1) * GROUP
NCHD = TOT // CH
DEG_G = NCHD // NW
NROWS = 10112
RPT = NROWS // NS
DEG_W = 16
H = 64


def _deg_body(dstc_hbm, ones_hbm, zeros_hbm, out_hbm, didx, onev, accum):
    c = lax.axis_index("c")
    s = lax.axis_index("s")
    wid = c * NS + s
    r0 = s * RPT
    pltpu.sync_copy(zeros_hbm.at[pl.ds(r0, RPT)], accum.at[pl.ds(r0, RPT)])
    pltpu.sync_copy(ones_hbm, onev)
    pltpu.sync_copy(dstc_hbm.at[pl.ds(wid * DEG_G, DEG_G)], didx)
    plsc.subcore_barrier()

    def body(j, carry):
        pltpu.sync_copy(onev, accum.at[didx.at[j]], add=True)
        return carry

    lax.fori_loop(0, DEG_G, body, 0)
    plsc.subcore_barrier()
    pltpu.sync_copy(accum.at[pl.ds(r0, RPT)], out_hbm.at[c, pl.ds(r0, RPT)])


def _scatter_body(src_hbm, dstc_hbm, vals_hbm, zeros_hbm, out_hbm,
                  sidx, didx, *rest):
    rows = rest[:NBUF]
    accum = rest[NBUF]
    vals_sh = rest[NBUF + 1]
    sems = rest[NBUF + 2:]
    c = lax.axis_index("c")
    s = lax.axis_index("s")
    r0 = s * RPT
    pltpu.sync_copy(zeros_hbm.at[pl.ds(r0, RPT)], accum.at[pl.ds(r0, RPT)])
    pltpu.sync_copy(vals_hbm.at[pl.ds(r0, RPT)], vals_sh.at[pl.ds(r0, RPT)])

    def gather(j, b):
        off = pl.multiple_of(j * GROUP, GROUP)
        return pltpu.async_copy(vals_sh.at[sidx.at[pl.ds(off, GROUP)]],
                                rows[b], sems[b])

    def gwait(b):
        pltpu.make_async_copy(zeros_hbm.at[pl.ds(0, GROUP)], rows[b],
                              sems[b]).wait()

    def scat(j, b):
        for k in range(GCH):
            pltpu.sync_copy(rows[b].at[pl.ds(k * CH, CH)],
                            accum.at[didx.at[j * GCH + k]], add=True)

    def run(gpr, reps, ebase, cbase):
        assert gpr % NBUF == 0
        for rep in range(reps):
            eb = ebase + rep * (gpr * GROUP)
            cb = cbase + rep * (gpr * GCH)
            n_ed = gpr * GROUP
            pltpu.sync_copy(
                src_hbm.at[pl.ds(pl.multiple_of(eb, GROUP), n_ed)],
                sidx.at[pl.ds(0, n_ed)])
            pltpu.sync_copy(dstc_hbm.at[pl.ds(cb, gpr * GCH)],
                            didx.at[pl.ds(0, gpr * GCH)])
            if rep == 0:
                plsc.subcore_barrier()
            for b in range(NBUF):
                gather(b, b)

            def body(i, carry):
                for b in range(NBUF):
                    j = i * NBUF + b
                    gwait(b)
                    scat(j, b)

                    @pl.when(j + NBUF < gpr)
                    def _():
                        gather(j + NBUF, b)
                return carry

            lax.fori_loop(0, gpr // NBUF, body, 0)
        plsc.subcore_barrier()
        pltpu.sync_copy(accum.at[pl.ds(r0, RPT)], out_hbm.at[c, pl.ds(r0, RPT)])

    @pl.when(c == 0)
    def _():
        run(GPR0, REPS0, s * (G0 * GROUP), s * (G0 * GCH))

    @pl.when(c == 1)
    def _():
        run(GPR1, REPS1, NS * (G0 * GROUP) + s * (G1 * GROUP),
            NS * (G0 * GCH) + s * (G1 * GCH))


@functools.lru_cache(maxsize=None)
def _build_sc_kernels():
    mesh = plsc.VectorSubcoreMesh(core_axis_name="c", subcore_axis_name="s",
                                  num_cores=NC, num_subcores=NS)
    cparams = pltpu.CompilerParams(use_tc_tiling_on_sc=False)
    deg_k = pl.kernel(
        _deg_body,
        out_type=jax.ShapeDtypeStruct((NC, NROWS, DEG_W), jnp.float32),
        mesh=mesh,
        scratch_types=[
            pltpu.VMEM((DEG_G, CH), jnp.int32),
            pltpu.VMEM((CH, DEG_W), jnp.float32),
            pltpu.VMEM_SHARED((NROWS, DEG_W), jnp.float32),
        ],
        compiler_params=cparams,
    )
    scat_k = pl.kernel(
        _scatter_body,
        out_type=jax.ShapeDtypeStruct((NC, NROWS, H), jnp.float32),
        mesh=mesh,
        scratch_types=[
            pltpu.VMEM((GPRMAX * GROUP,), jnp.int32),
            pltpu.VMEM((GPRMAX * GCH, CH), jnp.int32),
            *[pltpu.VMEM((GROUP, H), jnp.float32) for _ in range(NBUF)],
            pltpu.VMEM_SHARED((NROWS, H), jnp.float32),
            pltpu.VMEM_SHARED((NROWS, H), jnp.float32),
            *[pltpu.SemaphoreType.DMA for _ in range(NBUF)],
        ],
        compiler_params=cparams,
    )
    return deg_k, scat_k


def _seg_scatter(src_flat, dstc, vals, zeros_h):
    return _build_sc_kernels()[1](src_flat, dstc, vals, zeros_h)


def _deg_count(dstc, ones_deg, zeros_deg):
    return _build_sc_kernels()[0](dstc, ones_deg, zeros_deg)


def _mm1_body(x_ref, w_ref, da_ref, db_ref, o_ref):
    deg = da_ref[...] + db_ref[...] + 1.0
    dinv = lax.rsqrt(deg)
    o_ref[...] = jnp.dot(x_ref[...], w_ref[...],
                         preferred_element_type=jnp.float32) * dinv


def _mid_body(sa_ref, sb_ref, hs_ref, da_ref, db_ref, b1_ref, mt_ref,
              sqt_ref, tst_ref, al_ref, gh_ref, gy_ref, bh_ref, by_ref,
              w2h_ref, w2y_ref, o_ref):
    deg2 = da_ref[...] + db_ref[...]
    dinv = lax.rsqrt(deg2 + 1.0)
    h = jnp.maximum((sa_ref[...] + sb_ref[...] + hs_ref[...]) * dinv
                    + b1_ref[...], 0.0)
    hm = jnp.dot(h, mt_ref[...], preferred_element_type=jnp.float32)
    d_feat = (jnp.sum(h * h, axis=1, keepdims=True) - 2.0 * hm + sqt_ref[...])
    deg_n = deg2 / jnp.maximum(jnp.max(deg2), 1.0)
    d_struct = (deg_n - tst_ref[...]) ** 2
    alpha = al_ref[0, 0]
    y = (1.0 - alpha) * d_feat + alpha * d_struct
    eps = 1e-5
    muh = jnp.mean(h, axis=0, keepdims=True)
    varh = jnp.mean((h - muh) ** 2, axis=0, keepdims=True)
    zh = gh_ref[...] * (h - muh) / jnp.sqrt(varh + eps) + bh_ref[...]
    muy = jnp.mean(y, axis=0, keepdims=True)
    vary = jnp.mean((y - muy) ** 2, axis=0, keepdims=True)
    zy = gy_ref[...] * (y - muy) / jnp.sqrt(vary + eps) + by_ref[...]
    ts = jnp.dot(zh, w2h_ref[...], preferred_element_type=jnp.float32)
    ts = ts + jnp.dot(zy, w2y_ref[...], preferred_element_type=jnp.float32)
    o_ref[...] = ts * dinv


def _fin_body(sa_ref, sb_ref, ts_ref, da_ref, db_ref, b2_ref, wl_ref,
              bl_ref, out_ref, z_ref):
    deg = da_ref[...] + db_ref[...] + 1.0
    dinv = lax.rsqrt(deg)
    z = jnp.maximum((sa_ref[...] + sb_ref[...] + ts_ref[...]) * dinv
                    + b2_ref[...], 0.0)
    z_ref[...] = z
    out_ref[...] = jnp.dot(z, wl_ref[...],
                           preferred_element_type=jnp.float32) + bl_ref[...]


def _mm1_call(x, W1, da, db):
    return pl.pallas_call(
        _mm1_body,
        out_shape=jax.ShapeDtypeStruct((N_NODES, H), jnp.float32),
    )(x, W1, da, db)


def _mid_call(sa, sb, hs, da, db, b1, mt, sqt, tst, al, gh, gy, bh, by,
              w2h, w2y):
    return pl.pallas_call(
        _mid_body,
        out_shape=jax.ShapeDtypeStruct((N_NODES, H), jnp.float32),
    )(sa, sb, hs, da, db, b1, mt, sqt, tst, al, gh, gy, bh, by, w2h, w2y)


def _fin_call(sa, sb, ts, da, db, b2, Wl, bl):
    return pl.pallas_call(
        _fin_body,
        out_shape=(jax.ShapeDtypeStruct((N_NODES, 16), jnp.float32),
                   jax.ShapeDtypeStruct((N_NODES, H), jnp.float32)),
    )(sa, sb, ts, da, db, b2, Wl, bl)


def kernel(x, edge_index, W1, b1, Tf, TA, qw, alpha0, gamma, beta, W2, b2,
           Wl, bl):
    pad = TOT - N_EDGES
    src_flat = jnp.concatenate([edge_index[0],
                                jnp.zeros((pad,), edge_index.dtype)])
    dst = jnp.concatenate([edge_index[1],
                           jnp.full((pad,), N_NODES, edge_index.dtype)])
    dstc = dst.reshape(NCHD, CH)
    zeros_h = jnp.zeros((NROWS, H), jnp.float32)
    zeros_deg = jnp.zeros((NROWS, DEG_W), jnp.float32)
    ones_deg = jnp.ones((CH, DEG_W), jnp.float32)

    q = jax.nn.softmax(qw, axis=1)
    alpha = jax.nn.sigmoid(alpha0)
    mean_t = jnp.einsum('tk,tkd->td', q, Tf)
    sq_t = jnp.einsum('tk,tk->t', q, jnp.sum(Tf * Tf, axis=-1))
    t_struct = jnp.einsum('tk,tkl,tl->t', q, TA, q)

    degp = _deg_count(dstc, ones_deg, zeros_deg)
    da = degp[0, :N_NODES, 0:1]
    db = degp[1, :N_NODES, 0:1]

    hs0 = _mm1_call(x, W1, da, db)
    p1 = _seg_scatter(src_flat, dstc,
                      jnp.pad(hs0, ((0, NROWS - N_NODES), (0, 0))), zeros_h)
    ts = _mid_call(p1[0, :N_NODES], p1[1, :N_NODES], hs0, da, db,
                   b1.reshape(1, H), mean_t.T, sq_t.reshape(1, -1),
                   t_struct.reshape(1, -1), alpha.reshape(1, 1),
                   gamma[:H].reshape(1, H), gamma[H:].reshape(1, -1),
                   beta[:H].reshape(1, H), beta[H:].reshape(1, -1),
                   W2[:H], W2[H:])

    p2 = _seg_scatter(src_flat, dstc,
                      jnp.pad(ts, ((0, NROWS - N_NODES), (0, 0))), zeros_h)
    out, z2 = _fin_call(p2[0, :N_NODES], p2[1, :N_NODES], ts, da, db,
                        b2.reshape(1, H), Wl, bl.reshape(1, -1))
    return (out, z2)


# speedup vs baseline: 1.9609x; 1.0687x over previous
import functools

import jax
import jax.numpy as jnp
from jax import lax
from jax.experimental import pallas as pl
from jax.experimental.pallas import tpu as pltpu
from jax.experimental.pallas import tpu_sc as plsc

N_NODES = 10000
N_EDGES = 320000
NC = 2
NS = 16
NW = NC * NS
CH = 128
GCH = 2
GROUP = GCH * CH
GPR0 = 20
REPS0 = 2
GPR1 = 20
REPS1 = 2
G0 = GPR0 * REPS0
G1 = GPR1 * REPS1
NBUF = 2
GPRMAX = max(GPR0, GPR1)
TOT = NS * (G0 + G1) * GROUP
NCHD = TOT // CH
DEG_G = NCHD // NW
NROWS = 10112
RPT = NROWS // NS
DEG_W = 16
H = 64


def _deg_body(dstc_hbm, ones_hbm, zeros_hbm, out_hbm, didx, onev, accum):
    c = lax.axis_index("c")
    s = lax.axis_index("s")
    wid = c * NS + s
    r0 = s * RPT
    pltpu.sync_copy(zeros_hbm.at[pl.ds(r0, RPT)], accum.at[pl.ds(r0, RPT)])
    pltpu.sync_copy(ones_hbm, onev)
    pltpu.sync_copy(dstc_hbm.at[pl.ds(wid * DEG_G, DEG_G)], didx)
    plsc.subcore_barrier()

    def body(j, carry):
        pltpu.sync_copy(onev, accum.at[didx.at[j]], add=True)
        return carry

    lax.fori_loop(0, DEG_G, body, 0)
    plsc.subcore_barrier()
    pltpu.sync_copy(accum.at[pl.ds(r0, RPT)], out_hbm.at[c, pl.ds(r0, RPT)])


def _scatter_body(src_hbm, dstc_hbm, vals_hbm, zeros_hbm, out_hbm,
                  sidx, didx, *rest):
    rows = rest[:NBUF]
    accum = rest[NBUF]
    vals_sh = rest[NBUF + 1]
    sems = rest[NBUF + 2:]
    c = lax.axis_index("c")
    s = lax.axis_index("s")
    r0 = s * RPT
    pltpu.sync_copy(zeros_hbm.at[pl.ds(r0, RPT)], accum.at[pl.ds(r0, RPT)])
    pltpu.sync_copy(vals_hbm.at[pl.ds(r0, RPT)], vals_sh.at[pl.ds(r0, RPT)])

    def gather(j, b):
        off = pl.multiple_of(j * GROUP, GROUP)
        return pltpu.async_copy(vals_sh.at[sidx.at[pl.ds(off, GROUP)]],
                                rows[b], sems[b])

    def gwait(b):
        pltpu.make_async_copy(zeros_hbm.at[pl.ds(0, GROUP)], rows[b],
                              sems[b]).wait()

    def scat(j, b):
        for k in range(GCH):
            pltpu.sync_copy(rows[b].at[pl.ds(k * CH, CH)],
                            accum.at[didx.at[j * GCH + k]], add=True)

    def run(gpr, reps, ebase, cbase):
        assert gpr % NBUF == 0
        for rep in range(reps):
            eb = ebase + rep * (gpr * GROUP)
            cb = cbase + rep * (gpr * GCH)
            n_ed = gpr * GROUP
            pltpu.sync_copy(
                src_hbm.at[pl.ds(pl.multiple_of(eb, GROUP), n_ed)],
                sidx.at[pl.ds(0, n_ed)])
            pltpu.sync_copy(dstc_hbm.at[pl.ds(cb, gpr * GCH)],
                            didx.at[pl.ds(0, gpr * GCH)])
            if rep == 0:
                plsc.subcore_barrier()
            for b in range(NBUF):
                gather(b, b)

            def body(i, carry):
                for b in range(NBUF):
                    j = i * NBUF + b
                    gwait(b)
                    scat(j, b)

                    @pl.when(j + NBUF < gpr)
                    def _():
                        gather(j + NBUF, b)
                return carry

            lax.fori_loop(0, gpr // NBUF, body, 0)
        plsc.subcore_barrier()
        pltpu.sync_copy(accum.at[pl.ds(r0, RPT)], out_hbm.at[c, pl.ds(r0, RPT)])

    @pl.when(c == 0)
    def _():
        run(GPR0, REPS0, s * (G0 * GROUP), s * (G0 * GCH))

    @pl.when(c == 1)
    def _():
        run(GPR1, REPS1, NS * (G0 * GROUP) + s * (G1 * GROUP),
            NS * (G0 * GCH) + s * (G1 * GCH))


@functools.lru_cache(maxsize=None)
def _build_sc_kernels():
    mesh = plsc.VectorSubcoreMesh(core_axis_name="c", subcore_axis_name="s",
                                  num_cores=NC, num_subcores=NS)
    cparams = pltpu.CompilerParams(use_tc_tiling_on_sc=False)
    deg_k = pl.kernel(
        _deg_body,
        out_type=jax.ShapeDtypeStruct((NC, NROWS, DEG_W), jnp.float32),
        mesh=mesh,
        scratch_types=[
            pltpu.VMEM((DEG_G, CH), jnp.int32),
            pltpu.VMEM((CH, DEG_W), jnp.float32),
            pltpu.VMEM_SHARED((NROWS, DEG_W), jnp.float32),
        ],
        compiler_params=cparams,
    )
    scat_k = pl.kernel(
        _scatter_body,
        out_type=jax.ShapeDtypeStruct((NC, NROWS, H), jnp.float32),
        mesh=mesh,
        scratch_types=[
            pltpu.VMEM((GPRMAX * GROUP,), jnp.int32),
            pltpu.VMEM((GPRMAX * GCH, CH), jnp.int32),
            *[pltpu.VMEM((GROUP, H), jnp.float32) for _ in range(NBUF)],
            pltpu.VMEM_SHARED((NROWS, H), jnp.float32),
            pltpu.VMEM_SHARED((NROWS, H), jnp.float32),
            *[pltpu.SemaphoreType.DMA for _ in range(NBUF)],
        ],
        compiler_params=cparams,
    )
    return deg_k, scat_k


def _seg_scatter(src_flat, dstc, vals, zeros_h):
    return _build_sc_kernels()[1](src_flat, dstc, vals, zeros_h)


def _deg_count(dstc, ones_deg, zeros_deg):
    return _build_sc_kernels()[0](dstc, ones_deg, zeros_deg)


def _mm1_body(x_ref, w_ref, dp_ref, o_ref):
    deg = dp_ref[0, :N_NODES, 0:1] + dp_ref[1, :N_NODES, 0:1] + 1.0
    dinv = lax.rsqrt(deg)
    o_ref[:N_NODES, :] = jnp.dot(x_ref[...], w_ref[...],
                                 preferred_element_type=jnp.float32) * dinv


def _mid_body(p_ref, hs_ref, dp_ref, b1_ref, mt_ref,
              sqt_ref, tst_ref, al_ref, gh_ref, gy_ref, bh_ref, by_ref,
              w2h_ref, w2y_ref, o_ref):
    deg2 = dp_ref[0, :N_NODES, 0:1] + dp_ref[1, :N_NODES, 0:1]
    dinv = lax.rsqrt(deg2 + 1.0)
    h = jnp.maximum((p_ref[0, :N_NODES, :] + p_ref[1, :N_NODES, :]
                     + hs_ref[:N_NODES, :]) * dinv + b1_ref[...], 0.0)
    hm = jnp.dot(h, mt_ref[...], preferred_element_type=jnp.float32)
    d_feat = (jnp.sum(h * h, axis=1, keepdims=True) - 2.0 * hm + sqt_ref[...])
    deg_n = deg2 / jnp.maximum(jnp.max(deg2), 1.0)
    d_struct = (deg_n - tst_ref[...]) ** 2
    alpha = al_ref[0, 0]
    y = (1.0 - alpha) * d_feat + alpha * d_struct
    eps = 1e-5
    muh = jnp.mean(h, axis=0, keepdims=True)
    varh = jnp.mean((h - muh) ** 2, axis=0, keepdims=True)
    zh = gh_ref[...] * (h - muh) / jnp.sqrt(varh + eps) + bh_ref[...]
    muy = jnp.mean(y, axis=0, keepdims=True)
    vary = jnp.mean((y - muy) ** 2, axis=0, keepdims=True)
    zy = gy_ref[...] * (y - muy) / jnp.sqrt(vary + eps) + by_ref[...]
    ts = jnp.dot(zh, w2h_ref[...], preferred_element_type=jnp.float32)
    ts = ts + jnp.dot(zy, w2y_ref[...], preferred_element_type=jnp.float32)
    o_ref[:N_NODES, :] = ts * dinv


def _fin_body(p_ref, ts_ref, dp_ref, b2_ref, wl_ref,
              bl_ref, out_ref, z_ref):
    deg = dp_ref[0, :N_NODES, 0:1] + dp_ref[1, :N_NODES, 0:1] + 1.0
    dinv = lax.rsqrt(deg)
    z = jnp.maximum((p_ref[0, :N_NODES, :] + p_ref[1, :N_NODES, :]
                     + ts_ref[:N_NODES, :]) * dinv + b2_ref[...], 0.0)
    z_ref[...] = z
    out_ref[...] = jnp.dot(z, wl_ref[...],
                           preferred_element_type=jnp.float32) + bl_ref[...]


def _mm1_call(x, W1, degp):
    return pl.pallas_call(
        _mm1_body,
        out_shape=jax.ShapeDtypeStruct((NROWS, H), jnp.float32),
    )(x, W1, degp)


def _mid_call(p1, hs, degp, b1, mt, sqt, tst, al, gh, gy, bh, by,
              w2h, w2y):
    return pl.pallas_call(
        _mid_body,
        out_shape=jax.ShapeDtypeStruct((NROWS, H), jnp.float32),
    )(p1, hs, degp, b1, mt, sqt, tst, al, gh, gy, bh, by, w2h, w2y)


def _fin_call(p2, ts, degp, b2, Wl, bl):
    return pl.pallas_call(
        _fin_body,
        out_shape=(jax.ShapeDtypeStruct((N_NODES, 16), jnp.float32),
                   jax.ShapeDtypeStruct((N_NODES, H), jnp.float32)),
    )(p2, ts, degp, b2, Wl, bl)


def kernel(x, edge_index, W1, b1, Tf, TA, qw, alpha0, gamma, beta, W2, b2,
           Wl, bl):
    pad = TOT - N_EDGES
    src_flat = jnp.concatenate([edge_index[0],
                                jnp.zeros((pad,), edge_index.dtype)])
    dst = jnp.concatenate([edge_index[1],
                           jnp.full((pad,), N_NODES, edge_index.dtype)])
    dstc = dst.reshape(NCHD, CH)
    zeros_h = jnp.zeros((NROWS, H), jnp.float32)
    zeros_deg = jnp.zeros((NROWS, DEG_W), jnp.float32)
    ones_deg = jnp.ones((CH, DEG_W), jnp.float32)

    q = jax.nn.softmax(qw, axis=1)
    alpha = jax.nn.sigmoid(alpha0)
    mean_t = jnp.einsum('tk,tkd->td', q, Tf)
    sq_t = jnp.einsum('tk,tk->t', q, jnp.sum(Tf * Tf, axis=-1))
    t_struct = jnp.einsum('tk,tkl,tl->t', q, TA, q)

    degp = _deg_count(dstc, ones_deg, zeros_deg)

    hs0 = _mm1_call(x, W1, degp)
    p1 = _seg_scatter(src_flat, dstc, hs0, zeros_h)
    ts = _mid_call(p1, hs0, degp,
                   b1.reshape(1, H), mean_t.T, sq_t.reshape(1, -1),
                   t_struct.reshape(1, -1), alpha.reshape(1, 1),
                   gamma[:H].reshape(1, H), gamma[H:].reshape(1, -1),
                   beta[:H].reshape(1, H), beta[H:].reshape(1, -1),
                   W2[:H], W2[H:])

    p2 = _seg_scatter(src_flat, dstc, ts, zeros_h)
    out, z2 = _fin_call(p2, ts, degp,
                        b2.reshape(1, H), Wl, bl.reshape(1, -1))
    return (out, z2)
